# baseline (speedup 1.0000x reference)
import jax
import jax.numpy as jnp
from jax import lax
from jax.experimental import pallas as pl
from jax.experimental.pallas import tpu as pltpu

_BN_EPS = 1e-5


def _batch_body(x_ref, wd1, wp1_ref, bp1_ref, wd2, wp2_ref, bp2_ref,
                w1_ref, b1_ref, wh_ref, bh_ref, ww_ref, bw_ref,
                o_ref, xp1_ref, xp2_ref):
    H, W, C = x_ref.shape
    Wp = xp1_ref.shape[1]
    OFF = 8

    HC = 16 if H % 16 == 0 else H

    def dw_chunk(xp_ref, wd, h0, off, cast):
        xs = [xp_ref[h0:h0 + HC + 2, off - 1 + dx:off - 1 + dx + W, :]
              for dx in range(3)]
        if cast:
            xs = [v.astype(jnp.bfloat16) for v in xs]
        ps = []
        for dy in range(3):
            for dx in range(3):
                tap = wd[3 * dy + dx:3 * dy + dx + 1, :].reshape(1, 1, C)
                ps.append(xs[dx][dy:dy + HC, :, :] * tap)
        while len(ps) > 1:
            nxt = [ps[i] + ps[i + 1] for i in range(0, len(ps) - 1, 2)]
            if len(ps) % 2:
                nxt.append(ps[-1])
            ps = nxt
        return ps[0]

    def pw(acc, wp_ref):
        return jnp.dot(acc.reshape(HC * W, C), wp_ref[...],
                       preferred_element_type=jnp.float32).reshape(HC, W, C)

    xp1_ref[1:H + 1, OFF:OFF + W, :] = x_ref[...]
    bp1 = bp1_ref[...].reshape(1, 1, C)
    for h0 in range(0, H, HC):
        z1 = pw(dw_chunk(xp1_ref, wd1, h0, OFF, True), wp1_ref)
        xp2_ref[1 + h0:1 + h0 + HC, OFF:OFF + W, :] = jnp.maximum(z1 + bp1, 0.0)

    ph_parts = []
    pw_sum = jnp.zeros((W, C), jnp.float32)
    bp2 = bp2_ref[...].reshape(1, 1, C)
    for h0 in range(0, H, HC):
        z2 = jnp.maximum(pw(dw_chunk(xp2_ref, wd2, h0, OFF, True),
                            wp2_ref) + bp2, 0.0)
        ph_parts.append(jnp.mean(z2, axis=1))
        pw_sum = pw_sum + jnp.sum(z2, axis=0)
        xp1_ref[1 + h0:1 + h0 + HC, OFF:OFF + W, :] = z2
    ph = jnp.concatenate(ph_parts, axis=0)
    pw_ = pw_sum * (1.0 / H)

    y = jnp.concatenate([ph, pw_], axis=0)
    y1 = jnp.dot(y, w1_ref[...], preferred_element_type=jnp.float32,
                 precision=lax.Precision.HIGHEST) + b1_ref[...]
    y2 = y1 * (jnp.clip(y1 + 3.0, 0.0, 6.0) * (1.0 / 6.0))
    a_h = jax.nn.sigmoid(
        jnp.dot(y2[0:H, :], wh_ref[...], preferred_element_type=jnp.float32,
                precision=lax.Precision.HIGHEST) + bh_ref[...])
    a_w = jax.nn.sigmoid(
        jnp.dot(y2[H:H + W, :], ww_ref[...], preferred_element_type=jnp.float32,
                precision=lax.Precision.HIGHEST) + bw_ref[...])

    x2_3d = xp1_ref[1:H + 1, OFF:OFF + W, :]
    res = x2_3d * a_h[:, None, :] * a_w[None, :, :]
    o_ref[...] = jnp.swapaxes(res, 0, 1).astype(o_ref.dtype)


def _fused_kernel(x_ref, wd1_ref, wp1_ref, bp1_ref, wd2_ref, wp2_ref, bp2_ref,
                  w1_ref, b1_ref, wh_ref, bh_ref, ww_ref, bw_ref,
                  o_ref, xp1_ref, xp2_ref):
    G, H, W, C = x_ref.shape
    Wp = xp1_ref.shape[2]

    Wp2 = xp2_ref.shape[2]
    @pl.when(pl.program_id(0) == 0)
    def _():
        for g in range(G):
            xp1_ref[g, 0:1, :, :] = jnp.zeros((1, Wp, C), jnp.float32)
            xp1_ref[g, H + 1:H + 2, :, :] = jnp.zeros((1, Wp, C), jnp.float32)
            xp1_ref[g, :, 7:8, :] = jnp.zeros((H + 2, 1, C), jnp.float32)
            xp1_ref[g, :, 8 + W:9 + W, :] = jnp.zeros((H + 2, 1, C), jnp.float32)
            xp2_ref[g, 0:1, :, :] = jnp.zeros((1, Wp2, C), jnp.float32)
            xp2_ref[g, H + 1:H + 2, :, :] = jnp.zeros((1, Wp2, C), jnp.float32)
            xp2_ref[g, :, 7:8, :] = jnp.zeros((H + 2, 1, C), jnp.float32)
            xp2_ref[g, :, 8 + W:9 + W, :] = jnp.zeros((H + 2, 1, C), jnp.float32)

    wd1 = wd1_ref[...]
    wd2 = wd2_ref[...]
    for g in range(G):
        _batch_body(x_ref.at[g], wd1, wp1_ref, bp1_ref, wd2, wp2_ref, bp2_ref,
                    w1_ref, b1_ref, wh_ref, bh_ref, ww_ref, bw_ref,
                    o_ref.at[g], xp1_ref.at[g], xp2_ref.at[g])


def kernel(x, dw0, pw0, pb0, g0, be0, m0, v0, dw2, pw2, pb2, g2, be2, m2, v2,
           w1, b1, g1, be1, m1, v1, wh, bh, ww, bw):
    N, H, W, C = x.shape
    mip = w1.shape[1]

    s0 = g0 / jnp.sqrt(v0 + _BN_EPS)
    wp1f = (pw0 * s0[None, :]).astype(jnp.bfloat16)
    bp1f = (pb0 * s0 + be0 - m0 * s0).reshape(1, C).astype(jnp.float32)
    s2 = g2 / jnp.sqrt(v2 + _BN_EPS)
    wp2f = (pw2 * s2[None, :]).astype(jnp.bfloat16)
    bp2f = (pb2 * s2 + be2 - m2 * s2).reshape(1, C).astype(jnp.float32)
    s1 = g1 / jnp.sqrt(v1 + _BN_EPS)
    w1f = (w1 * s1[None, :]).astype(jnp.float32)
    b1f = ((b1 - m1) * s1 + be1).reshape(1, mip).astype(jnp.float32)

    wd1 = dw0.reshape(9, C).astype(jnp.bfloat16)
    wd2 = dw2.reshape(9, C).astype(jnp.bfloat16)
    bh2 = bh.reshape(1, C).astype(jnp.float32)
    bw2 = bw.reshape(1, C).astype(jnp.float32)

    G = 2 if N % 2 == 0 else 1
    full = lambda shape: pl.BlockSpec(shape, lambda n: tuple(0 for _ in shape))
    out = pl.pallas_call(
        _fused_kernel,
        out_shape=jax.ShapeDtypeStruct((N, W, H, C), x.dtype),
        grid=(N // G,),
        in_specs=[
            pl.BlockSpec((G, H, W, C), lambda n: (n, 0, 0, 0)),
            full((9, C)), full((C, C)), full((1, C)),
            full((9, C)), full((C, C)), full((1, C)),
            full((C, mip)), full((1, mip)),
            full((mip, C)), full((1, C)),
            full((mip, C)), full((1, C)),
        ],
        out_specs=pl.BlockSpec((G, W, H, C), lambda n: (n, 0, 0, 0)),
        scratch_shapes=[
            pltpu.VMEM((G, H + 2, W + 16, C), jnp.float32),
            pltpu.VMEM((G, H + 2, W + 16, C), jnp.float32),
        ],
        compiler_params=pltpu.CompilerParams(
            dimension_semantics=("parallel",),
            vmem_limit_bytes=48 * 1024 * 1024,
        ),
    )(x, wd1, wp1f, bp1f, wd2, wp2f, bp2f, w1f, b1f,
      wh.astype(jnp.float32), bh2, ww.astype(jnp.float32), bw2)

    return out


# speedup vs baseline: 1.0024x; 1.0024x over previous
import jax
import jax.numpy as jnp
from jax import lax
from jax.experimental import pallas as pl
from jax.experimental.pallas import tpu as pltpu

_BN_EPS = 1e-5


def _batch_body(x_ref, wd1, wp1_ref, bp1_ref, wd2, wp2_ref, bp2_ref,
                w1_ref, b1_ref, wh_ref, bh_ref, ww_ref, bw_ref,
                o_ref, xp1_ref, xp2_ref):
    H, W, C = x_ref.shape
    Wp = xp1_ref.shape[1]
    OFF = 8

    HC = 16 if H % 16 == 0 else H

    def dw_chunk(xp_ref, wd, h0):
        xs = [xp_ref[h0:h0 + HC + 2, OFF - 1 + dx:OFF - 1 + dx + W, :]
              .astype(jnp.bfloat16) for dx in range(3)]
        ps = []
        for dy in range(3):
            for dx in range(3):
                tap = wd[3 * dy + dx:3 * dy + dx + 1, :].reshape(1, 1, C)
                ps.append(xs[dx][dy:dy + HC, :, :] * tap)
        while len(ps) > 1:
            nxt = [ps[i] + ps[i + 1] for i in range(0, len(ps) - 1, 2)]
            if len(ps) % 2:
                nxt.append(ps[-1])
            ps = nxt
        return ps[0]

    def pw(acc, wp_ref):
        return jnp.dot(acc.reshape(HC * W, C), wp_ref[...],
                       preferred_element_type=jnp.float32).reshape(HC, W, C)

    xp1_ref[1:H + 1, OFF:OFF + W, :] = x_ref[...]
    bp1 = bp1_ref[...].reshape(1, 1, C)
    for h0 in range(0, H, HC):
        z1 = pw(dw_chunk(xp1_ref, wd1, h0), wp1_ref)
        xp2_ref[1 + h0:1 + h0 + HC, OFF:OFF + W, :] = jnp.maximum(z1 + bp1, 0.0)

    ph_parts = []
    pw_sum = jnp.zeros((W, C), jnp.float32)
    bp2 = bp2_ref[...].reshape(1, 1, C)
    for h0 in range(0, H, HC):
        z2 = jnp.maximum(pw(dw_chunk(xp2_ref, wd2, h0), wp2_ref) + bp2, 0.0)
        ph_parts.append(jnp.mean(z2, axis=1))
        pw_sum = pw_sum + jnp.sum(z2, axis=0)
        xp1_ref[1 + h0:1 + h0 + HC, OFF:OFF + W, :] = z2
    ph = jnp.concatenate(ph_parts, axis=0)
    pw_ = pw_sum * (1.0 / H)

    y = jnp.concatenate([ph, pw_], axis=0)
    y1 = jnp.dot(y, w1_ref[...], preferred_element_type=jnp.float32,
                 precision=lax.Precision.HIGHEST) + b1_ref[...]
    y2 = y1 * (jnp.clip(y1 + 3.0, 0.0, 6.0) * (1.0 / 6.0))
    a_h = jax.nn.sigmoid(
        jnp.dot(y2[0:H, :], wh_ref[...], preferred_element_type=jnp.float32,
                precision=lax.Precision.HIGHEST) + bh_ref[...])
    a_w = jax.nn.sigmoid(
        jnp.dot(y2[H:H + W, :], ww_ref[...], preferred_element_type=jnp.float32,
                precision=lax.Precision.HIGHEST) + bw_ref[...])

    x2_3d = xp1_ref[1:H + 1, OFF:OFF + W, :]
    res = x2_3d * a_h[:, None, :] * a_w[None, :, :]
    o_ref[...] = jnp.swapaxes(res, 0, 1).astype(o_ref.dtype)


def _fused_kernel(x_ref, wd1_ref, wp1_ref, bp1_ref, wd2_ref, wp2_ref, bp2_ref,
                  w1_ref, b1_ref, wh_ref, bh_ref, ww_ref, bw_ref,
                  o_ref, xp1_ref, xp2_ref):
    G, H, W, C = x_ref.shape
    Wp = xp1_ref.shape[2]

    Wp2 = xp2_ref.shape[2]
    @pl.when(pl.program_id(0) == 0)
    def _():
        for g in range(G):
            xp1_ref[g, 0:1, :, :] = jnp.zeros((1, Wp, C), jnp.float32)
            xp1_ref[g, H + 1:H + 2, :, :] = jnp.zeros((1, Wp, C), jnp.float32)
            xp1_ref[g, :, 7:8, :] = jnp.zeros((H + 2, 1, C), jnp.float32)
            xp1_ref[g, :, 8 + W:9 + W, :] = jnp.zeros((H + 2, 1, C), jnp.float32)
            xp2_ref[g, 0:1, :, :] = jnp.zeros((1, Wp2, C), jnp.float32)
            xp2_ref[g, H + 1:H + 2, :, :] = jnp.zeros((1, Wp2, C), jnp.float32)
            xp2_ref[g, :, 7:8, :] = jnp.zeros((H + 2, 1, C), jnp.float32)
            xp2_ref[g, :, 8 + W:9 + W, :] = jnp.zeros((H + 2, 1, C), jnp.float32)

    wd1 = wd1_ref[...]
    wd2 = wd2_ref[...]
    for g in range(G):
        _batch_body(x_ref.at[g], wd1, wp1_ref, bp1_ref, wd2, wp2_ref, bp2_ref,
                    w1_ref, b1_ref, wh_ref, bh_ref, ww_ref, bw_ref,
                    o_ref.at[g], xp1_ref.at[g], xp2_ref.at[g])


def kernel(x, dw0, pw0, pb0, g0, be0, m0, v0, dw2, pw2, pb2, g2, be2, m2, v2,
           w1, b1, g1, be1, m1, v1, wh, bh, ww, bw):
    N, H, W, C = x.shape
    mip = w1.shape[1]

    s0 = g0 / jnp.sqrt(v0 + _BN_EPS)
    wp1f = (pw0 * s0[None, :]).astype(jnp.bfloat16)
    bp1f = (pb0 * s0 + be0 - m0 * s0).reshape(1, C).astype(jnp.float32)
    s2 = g2 / jnp.sqrt(v2 + _BN_EPS)
    wp2f = (pw2 * s2[None, :]).astype(jnp.bfloat16)
    bp2f = (pb2 * s2 + be2 - m2 * s2).reshape(1, C).astype(jnp.float32)
    s1 = g1 / jnp.sqrt(v1 + _BN_EPS)
    w1f = (w1 * s1[None, :]).astype(jnp.float32)
    b1f = ((b1 - m1) * s1 + be1).reshape(1, mip).astype(jnp.float32)

    wd1 = dw0.reshape(9, C).astype(jnp.bfloat16)
    wd2 = dw2.reshape(9, C).astype(jnp.bfloat16)
    bh2 = bh.reshape(1, C).astype(jnp.float32)
    bw2 = bw.reshape(1, C).astype(jnp.float32)

    G = 2 if N % 2 == 0 else 1
    full = lambda shape: pl.BlockSpec(shape, lambda n: tuple(0 for _ in shape))
    out = pl.pallas_call(
        _fused_kernel,
        out_shape=jax.ShapeDtypeStruct((N, W, H, C), x.dtype),
        grid=(N // G,),
        in_specs=[
            pl.BlockSpec((G, H, W, C), lambda n: (n, 0, 0, 0)),
            full((9, C)), full((C, C)), full((1, C)),
            full((9, C)), full((C, C)), full((1, C)),
            full((C, mip)), full((1, mip)),
            full((mip, C)), full((1, C)),
            full((mip, C)), full((1, C)),
        ],
        out_specs=pl.BlockSpec((G, W, H, C), lambda n: (n, 0, 0, 0)),
        scratch_shapes=[
            pltpu.VMEM((G, H + 2, W + 16, C), jnp.float32),
            pltpu.VMEM((G, H + 2, W + 16, C), jnp.float32),
        ],
        compiler_params=pltpu.CompilerParams(
            dimension_semantics=("parallel",),
            vmem_limit_bytes=48 * 1024 * 1024,
        ),
    )(x, wd1, wp1f, bp1f, wd2, wp2f, bp2f, w1f, b1f,
      wh.astype(jnp.float32), bh2, ww.astype(jnp.float32), bw2)

    return out
